# baseline (speedup 1.0000x reference)
import functools

import jax
import jax.numpy as jnp
from jax import lax
from jax.experimental import pallas as pl
from jax.experimental.pallas import tpu as pltpu
from jax.experimental.pallas import tpu_sc as plsc

NC = 2
NS = 16
L = 16
NW = NC * NS
CH = 128


def _make_kernel(B, D):
    b_per_w = B // NW
    n_chunk = b_per_w // CH
    mesh = plsc.VectorSubcoreMesh(core_axis_name="c", subcore_axis_name="s")

    @functools.partial(
        pl.kernel,
        out_type=jax.ShapeDtypeStruct((B,), jnp.float32),
        mesh=mesh,
        compiler_params=pltpu.CompilerParams(needs_layout_passes=False, use_tc_tiling_on_sc=False),
        scratch_types=[
            pltpu.VMEM((n_chunk, CH), jnp.int32),
            pltpu.VMEM((n_chunk, CH), jnp.int32),
            pltpu.VMEM((b_per_w, D), jnp.float32),
            pltpu.VMEM((b_per_w, D), jnp.float32),
            pltpu.VMEM((D, L), jnp.float32),
            pltpu.VMEM((L,), jnp.float32),
            pltpu.VMEM((b_per_w,), jnp.float32),
            pltpu.SemaphoreType.DMA,
        ],
    )
    def kern(user2d, song2d, uemb, semb, wtab_h, bvec_h, out_h,
             uidx, sidx, urows, srows, wtab, bvec, out_v, sem):
        wid = lax.axis_index("s") * NC + lax.axis_index("c")
        cbase = wid * n_chunk

        pltpu.sync_copy(user2d.at[pl.ds(cbase, n_chunk)], uidx)
        pltpu.sync_copy(song2d.at[pl.ds(cbase, n_chunk)], sidx)
        pltpu.sync_copy(wtab_h, wtab)
        pltpu.sync_copy(bvec_h, bvec)

        copies = []
        for k in range(n_chunk):
            copies.append(pltpu.async_copy(
                uemb.at[uidx.at[k]], urows.at[pl.ds(k * CH, CH)], sem))
            copies.append(pltpu.async_copy(
                semb.at[sidx.at[k]], srows.at[pl.ds(k * CH, CH)], sem))
        for c in copies:
            c.wait()

        def g_body(g, carry):
            rowv = lax.iota(jnp.int32, L) + g * L
            acc = bvec[:]
            for d in range(D):
                w = wtab[d, :]
                col = jnp.full((L,), d, jnp.int32)
                u = plsc.load_gather(urows, [rowv, col])
                s = plsc.load_gather(srows, [rowv, col])
                acc = acc + u * s * w
            out_v[pl.ds(g * L, L)] = acc
            return carry

        lax.fori_loop(0, b_per_w // L, g_body, 0)
        pltpu.sync_copy(out_v, out_h.at[pl.ds(wid * b_per_w, b_per_w)])

    return kern


def kernel(user, song, user_embedding, song_embedding, fc_w, fc_b):
    B = user.shape[0]
    D = user_embedding.shape[1]
    kern = _make_kernel(B, D)
    user2d = user.astype(jnp.int32).reshape(B // CH, CH)
    song2d = song.astype(jnp.int32).reshape(B // CH, CH)
    wtab = jnp.broadcast_to(fc_w.reshape(D, 1), (D, L)).astype(jnp.float32)
    bvec = jnp.broadcast_to(fc_b.reshape(1), (L,)).astype(jnp.float32)
    return kern(user2d, song2d, user_embedding, song_embedding, wtab, bvec)


# speedup vs baseline: 1.0021x; 1.0021x over previous
import functools

import jax
import jax.numpy as jnp
from jax import lax
from jax.experimental import pallas as pl
from jax.experimental.pallas import tpu as pltpu
from jax.experimental.pallas import tpu_sc as plsc

NC = 2
NS = 16
L = 16
NW = NC * NS
CH = 128


def _make_kernel(B, D):
    b_per_w = B // NW
    n_chunk = b_per_w // CH
    D2 = 2 * D
    mesh = plsc.VectorSubcoreMesh(core_axis_name="c", subcore_axis_name="s")

    @functools.partial(
        pl.kernel,
        out_type=jax.ShapeDtypeStruct((B,), jnp.float32),
        mesh=mesh,
        compiler_params=pltpu.CompilerParams(
            needs_layout_passes=False, use_tc_tiling_on_sc=True
        ),
        scratch_types=[
            pltpu.VMEM((n_chunk, CH), jnp.int32),
            pltpu.VMEM((n_chunk, CH), jnp.int32),
            pltpu.VMEM((n_chunk, CH), jnp.int32),
            pltpu.VMEM((n_chunk, CH), jnp.int32),
            pltpu.VMEM((2, CH, D2), jnp.float32),
            pltpu.VMEM((2, CH, D2), jnp.float32),
            pltpu.VMEM((D, L), jnp.float32),
            pltpu.VMEM((L,), jnp.float32),
            pltpu.VMEM((b_per_w,), jnp.float32),
            pltpu.SemaphoreType.DMA,
            pltpu.SemaphoreType.DMA,
        ],
    )
    def kern(user2d, song2d, upar2d, spar2d, uemb2, semb2, wtab_h, bvec_h,
             out_h, uidx, sidx, upar, spar, ublk, sblk, wtab, bvec, out_v,
             sem0, sem1):
        wid = lax.axis_index("s") * NC + lax.axis_index("c")
        cbase = wid * n_chunk

        pltpu.sync_copy(user2d.at[pl.ds(cbase, n_chunk)], uidx)
        pltpu.sync_copy(song2d.at[pl.ds(cbase, n_chunk)], sidx)
        pltpu.sync_copy(upar2d.at[pl.ds(cbase, n_chunk)], upar)
        pltpu.sync_copy(spar2d.at[pl.ds(cbase, n_chunk)], spar)
        pltpu.sync_copy(wtab_h, wtab)
        pltpu.sync_copy(bvec_h, bvec)

        sems = (sem0, sem1)

        def fire(k):
            buf = k % 2
            pltpu.async_copy(uemb2.at[uidx.at[k]], ublk.at[buf], sems[buf])
            pltpu.async_copy(semb2.at[sidx.at[k]], sblk.at[buf], sems[buf])

        fire(0)
        fire(1)

        for k in range(n_chunk):
            buf = k % 2
            pltpu.make_async_copy(
                uemb2.at[uidx.at[k]], ublk.at[buf], sems[buf]).wait()
            pltpu.make_async_copy(
                semb2.at[sidx.at[k]], sblk.at[buf], sems[buf]).wait()

            def g_body(g, carry, k=k, buf=buf):
                rowv = lax.iota(jnp.int32, L) + g * L
                pu = upar[k, pl.ds(g * L, L)]
                ps = spar[k, pl.ds(g * L, L)]
                acc = bvec[:]
                for d in range(D):
                    w = wtab[d, :]
                    u = plsc.load_gather(ublk.at[buf], [rowv, pu + d])
                    s = plsc.load_gather(sblk.at[buf], [rowv, ps + d])
                    acc = acc + u * s * w
                out_v[pl.ds(k * CH + g * L, L)] = acc
                return carry

            lax.fori_loop(0, CH // L, g_body, 0)
            if k + 2 < n_chunk:
                fire(k + 2)
        pltpu.sync_copy(out_v, out_h.at[pl.ds(wid * b_per_w, b_per_w)])

    return kern


def kernel(user, song, user_embedding, song_embedding, fc_w, fc_b):
    B = user.shape[0]
    N, D = user_embedding.shape
    kern = _make_kernel(B, D)
    user = user.astype(jnp.int32)
    song = song.astype(jnp.int32)
    user2d = (user // 2).reshape(B // CH, CH)
    song2d = (song // 2).reshape(B // CH, CH)
    upar2d = ((user % 2) * D).reshape(B // CH, CH)
    spar2d = ((song % 2) * D).reshape(B // CH, CH)
    wtab = jnp.broadcast_to(fc_w.reshape(D, 1), (D, L)).astype(jnp.float32)
    bvec = jnp.broadcast_to(fc_b.reshape(1), (L,)).astype(jnp.float32)
    return kern(user2d, song2d, upar2d, spar2d,
                user_embedding.reshape(N // 2, 2 * D),
                song_embedding.reshape(N // 2, 2 * D),
                wtab, bvec)


# speedup vs baseline: 2.7067x; 2.7010x over previous
import functools

import jax
import jax.numpy as jnp
from jax import lax
from jax.experimental import pallas as pl
from jax.experimental.pallas import tpu as pltpu
from jax.experimental.pallas import tpu_sc as plsc

NC = 2
NS = 16
L = 16
NW = NC * NS
NBUF = 4
TW = 128


def _make_kernel(B, D):
    b_per_w = B // NW
    mesh = plsc.VectorSubcoreMesh(core_axis_name="c", subcore_axis_name="s")

    @functools.partial(
        pl.kernel,
        out_type=jax.ShapeDtypeStruct((B,), jnp.float32),
        mesh=mesh,
        compiler_params=pltpu.CompilerParams(
            needs_layout_passes=False, use_tc_tiling_on_sc=True
        ),
        scratch_types=[
            pltpu.VMEM((b_per_w,), jnp.int32),
            pltpu.VMEM((b_per_w,), jnp.int32),
            pltpu.VMEM((NBUF, D, TW), jnp.float32),
            pltpu.VMEM((NBUF, D, TW), jnp.float32),
            pltpu.VMEM((D,), jnp.float32),
            pltpu.VMEM((L,), jnp.float32),
            pltpu.VMEM((b_per_w,), jnp.float32),
        ]
        + [pltpu.SemaphoreType.DMA] * NBUF,
    )
    def kern(user_h, song_h, uembT, sembT, w_h, bvec_h, out_h,
             uidx_v, sidx_v, ublk, sblk, wv, bvec, out_v,
             *sems):
        wid = lax.axis_index("s") * NC + lax.axis_index("c")
        base = wid * b_per_w

        pltpu.sync_copy(user_h.at[pl.ds(base, b_per_w)], uidx_v)
        pltpu.sync_copy(song_h.at[pl.ds(base, b_per_w)], sidx_v)
        pltpu.sync_copy(w_h, wv)
        pltpu.sync_copy(bvec_h, bvec)

        wregs = [wv[pl.ds(j * L, L)] for j in range(D // L)]
        lane_iota = lax.iota(jnp.int32, L)

        def idx_scalar(ref, e):
            chunk = ref[pl.ds((e // L) * L, L)]
            return jnp.sum(jnp.where(lane_iota == e % L, chunk, 0))

        def fetch(e, slot):
            ru = idx_scalar(uidx_v, e)
            rs = idx_scalar(sidx_v, e)
            cu = pl.multiple_of((ru // TW) * TW, TW)
            cs = pl.multiple_of((rs // TW) * TW, TW)
            pltpu.async_copy(uembT.at[:, pl.ds(cu, TW)], ublk.at[slot],
                             sems[slot])
            pltpu.async_copy(sembT.at[:, pl.ds(cs, TW)], sblk.at[slot],
                             sems[slot])

        for s in range(NBUF):
            fetch(s, s)

        def body(g, acc):
            for slot in range(NBUF):
                e = g * NBUF + slot
                pltpu.make_async_copy(uembT.at[:, pl.ds(0, TW)],
                                      ublk.at[slot], sems[slot]).wait()
                pltpu.make_async_copy(sembT.at[:, pl.ds(0, TW)],
                                      sblk.at[slot], sems[slot]).wait()
                lane_u = jnp.broadcast_to(idx_scalar(uidx_v, e) % TW, (L,))
                lane_s = jnp.broadcast_to(idx_scalar(sidx_v, e) % TW, (L,))
                p = jnp.zeros((L,), jnp.float32)
                for j in range(D // L):
                    rows = lane_iota + j * L
                    u = plsc.load_gather(ublk.at[slot], [rows, lane_u])
                    s = plsc.load_gather(sblk.at[slot], [rows, lane_s])
                    p = p + u * s * wregs[j]
                val = jnp.sum(p)

                @pl.when(e + NBUF < b_per_w)
                def _():
                    fetch(e + NBUF, slot)

                acc = jnp.where(lane_iota == e % L, val, acc)

                @pl.when(e % L == L - 1)
                def _():
                    out_v[pl.ds((e // L) * L, L)] = acc + bvec[:]
                acc = jnp.where(e % L == L - 1,
                                jnp.zeros((L,), jnp.float32), acc)
            return acc

        lax.fori_loop(0, b_per_w // NBUF, body,
                      jnp.zeros((L,), jnp.float32))
        pltpu.sync_copy(out_v, out_h.at[pl.ds(base, b_per_w)])

    return kern


def kernel(user, song, user_embedding, song_embedding, fc_w, fc_b):
    B = user.shape[0]
    D = user_embedding.shape[1]
    kern = _make_kernel(B, D)
    return kern(
        user.astype(jnp.int32),
        song.astype(jnp.int32),
        user_embedding.T,
        song_embedding.T,
        fc_w.reshape(D).astype(jnp.float32),
        jnp.broadcast_to(fc_b.reshape(1), (L,)).astype(jnp.float32),
    )
